# baseline (speedup 1.0000x reference)
import functools

import jax
import jax.numpy as jnp
from jax import lax
from jax.experimental import pallas as pl
from jax.experimental.pallas import tpu as pltpu
from jax.experimental.pallas import tpu_sc as plsc

N = 10000
E = 160000
D = 128
H = 32

NC = 2
NS = 16
NW = NC * NS
CHUNK = 128
CPT = -(-E // (CHUNK * NW))
NCHUNKS_PAD = CPT * NW
E_PAD = NCHUNKS_PAD * CHUNK
N8 = N + 8
NPAD = 10016
STRIPE = 624
TAIL = N - NS * STRIPE

_mesh = plsc.VectorSubcoreMesh(core_axis_name="c", subcore_axis_name="s")


@functools.partial(
    pl.kernel,
    out_type=(
        jax.ShapeDtypeStruct((NC, 1, NPAD), jnp.float32),
        jax.ShapeDtypeStruct((NC, 1, NPAD), jnp.float32),
    ),
    mesh=_mesh,
    scratch_types=[
        pltpu.VMEM((CPT, CHUNK), jnp.int32),
        pltpu.VMEM((CPT, CHUNK), jnp.int32),
        pltpu.VMEM((CHUNK,), jnp.float32),
        pltpu.VMEM((NPAD,), jnp.float32),
        pltpu.VMEM_SHARED((NPAD,), jnp.float32),
        pltpu.VMEM_SHARED((NPAD,), jnp.float32),
        pltpu.SemaphoreType.DMA,
        pltpu.SemaphoreType.DMA,
    ],
)
def _deg_call(src_hbm, dst_hbm, dego_hbm, degi_hbm, src_idx, dst_idx, ones_v,
              zeros_v, sh_dego, sh_degi, sem_a, sem_b):
    cid = lax.axis_index("c")
    sid = lax.axis_index("s")
    wid = cid * NS + sid

    pltpu.sync_copy(src_hbm.at[pl.ds(wid * CPT, CPT)], src_idx)
    pltpu.sync_copy(dst_hbm.at[pl.ds(wid * CPT, CPT)], dst_idx)

    for i in range(CHUNK // 16):
        ones_v[pl.ds(i * 16, 16)] = jnp.ones((16,), jnp.float32)

    @pl.when(sid == 0)
    def _init():
        def zbody(i, _):
            zeros_v[pl.ds(i * 16, 16)] = jnp.zeros((16,), jnp.float32)
            return ()
        lax.fori_loop(0, NPAD // 16, zbody, ())
        pltpu.sync_copy(zeros_v, sh_dego)
        pltpu.sync_copy(zeros_v, sh_degi)

    plsc.subcore_barrier()

    def dbody(j, _):
        pltpu.sync_copy(ones_v, sh_dego.at[src_idx.at[j]], add=True)
        pltpu.sync_copy(ones_v, sh_degi.at[dst_idx.at[j]], add=True)
        return ()

    lax.fori_loop(0, CPT, dbody, ())

    plsc.subcore_barrier()

    @pl.when(sid == 0)
    def _writeout():
        pltpu.sync_copy(sh_dego, dego_hbm.at[cid, 0])
        pltpu.sync_copy(sh_degi, degi_hbm.at[cid, 0])


@functools.partial(
    pl.kernel,
    out_type=jax.ShapeDtypeStruct((NC, N, D), jnp.float32),
    mesh=_mesh,
    scratch_types=[
        pltpu.VMEM((CPT, CHUNK), jnp.int32),
        pltpu.VMEM((CPT, CHUNK), jnp.int32),
        pltpu.VMEM((CHUNK, D), jnp.float32),
        pltpu.VMEM((CHUNK, D), jnp.float32),
        pltpu.VMEM((16, D), jnp.float32),
        pltpu.VMEM_SHARED((N8, D), jnp.float32),
        pltpu.SemaphoreType.DMA,
        pltpu.SemaphoreType.DMA,
        pltpu.SemaphoreType.DMA,
        pltpu.SemaphoreType.DMA,
    ],
)
def _agg_call(src_hbm, dst_hbm, xs_hbm, agg_hbm, src_idx, dst_idx, rows0,
              rows1, zrow_v, sh_agg, gsem0, gsem1, ssem0, ssem1):
    cid = lax.axis_index("c")
    sid = lax.axis_index("s")
    wid = cid * NS + sid

    pltpu.sync_copy(src_hbm.at[pl.ds(wid * CPT, CPT)], src_idx)
    pltpu.sync_copy(dst_hbm.at[pl.ds(wid * CPT, CPT)], dst_idx)

    def zbody(i, _):
        for j in range(D // 16):
            zrow_v[i, pl.ds(j * 16, 16)] = jnp.zeros((16,), jnp.float32)
        return ()
    lax.fori_loop(0, 16, zbody, ())

    def zcopy(k, _):
        pltpu.sync_copy(zrow_v, sh_agg.at[pl.ds(sid * STRIPE + k * 16, 16)])
        return ()
    lax.fori_loop(0, STRIPE // 16, zcopy, ())

    @pl.when(sid == 0)
    def _init_tail():
        pltpu.sync_copy(zrow_v, sh_agg.at[pl.ds(NS * STRIPE, TAIL)])

    plsc.subcore_barrier()

    def body(k, _):
        j0 = k * 2
        j1 = k * 2 + 1
        g0 = pltpu.async_copy(xs_hbm.at[src_idx.at[j0]], rows0, gsem0)
        g1 = pltpu.async_copy(xs_hbm.at[src_idx.at[j1]], rows1, gsem1)
        g0.wait()
        pltpu.sync_copy(rows0, sh_agg.at[dst_idx.at[j0]], add=True)
        g1.wait()
        pltpu.sync_copy(rows1, sh_agg.at[dst_idx.at[j1]], add=True)
        return ()

    lax.fori_loop(0, CPT // 2, body, ())

    plsc.subcore_barrier()

    row0 = sid * STRIPE
    pltpu.sync_copy(sh_agg.at[pl.ds(row0, STRIPE)],
                    agg_hbm.at[cid, pl.ds(row0, STRIPE)])

    @pl.when(sid == 0)
    def _writeout_tail():
        pltpu.sync_copy(sh_agg.at[pl.ds(NS * STRIPE, TAIL)],
                        agg_hbm.at[cid, pl.ds(NS * STRIPE, TAIL)])


def _xs_body(x_ref, dego_ref, xs_ref):
    d = dego_ref[0, 0, pl.ds(0, N)] + dego_ref[1, 0, pl.ds(0, N)]
    norm = jnp.where(d > 0, lax.rsqrt(jnp.maximum(d, 1.0)), 0.0)
    xs_ref[pl.ds(0, N), :] = x_ref[...] * norm[:, None]
    xs_ref[pl.ds(N, 8), :] = jnp.zeros((8, D), jnp.float32)


def _xs_call(x, dego):
    return pl.pallas_call(
        _xs_body,
        out_shape=jax.ShapeDtypeStruct((N8, D), jnp.float32),
    )(x, dego)


def _z_body(agg_ref, w_ref, degi_ref, b_ref, z_ref):
    a = agg_ref[0] + agg_ref[1]
    d = degi_ref[0, 0, pl.ds(0, N)] + degi_ref[1, 0, pl.ds(0, N)]
    norm = jnp.where(d > 0, lax.rsqrt(jnp.maximum(d, 1.0)), 0.0)
    zw = jnp.dot(a, w_ref[...], preferred_element_type=jnp.float32)
    z_ref[...] = zw * norm[:, None] + b_ref[0, :][None, :]


def _z_call(aggp, W, degi, b2d):
    return pl.pallas_call(
        _z_body,
        out_shape=jax.ShapeDtypeStruct((N, H), jnp.float32),
    )(aggp, W, degi, b2d)


BI = 256
NBI = (N + BI - 1) // BI


def _recon_body(zi_ref, zall_ref, out_ref):
    out_ref[...] = lax.dot_general(
        zi_ref[...], zall_ref[...],
        dimension_numbers=(((1,), (1,)), ((), ())),
        preferred_element_type=jnp.float32,
    )


def _recon_call(z):
    return pl.pallas_call(
        _recon_body,
        grid=(NBI,),
        in_specs=[
            pl.BlockSpec((BI, H), lambda i: (i, 0)),
            pl.BlockSpec((N, H), lambda i: (0, 0)),
        ],
        out_specs=pl.BlockSpec((BI, N), lambda i: (i, 0)),
        out_shape=jax.ShapeDtypeStruct((N, N), jnp.float32),
    )(z, z)


def kernel(x, edge_index, W, b):
    npad = E_PAD - E
    pad_deg = N + (jnp.arange(npad, dtype=jnp.int32) % 16)
    pad_gat = jnp.full((npad,), N, jnp.int32)
    pad_sct = jnp.arange(npad, dtype=jnp.int32) % N
    src = edge_index[0]
    dst = edge_index[1]
    srcp_d = jnp.concatenate([src, pad_deg]).reshape(NCHUNKS_PAD, CHUNK)
    dstp_d = jnp.concatenate([dst, pad_deg]).reshape(NCHUNKS_PAD, CHUNK)
    srcp_a = jnp.concatenate([src, pad_gat]).reshape(NCHUNKS_PAD, CHUNK)
    dstp_a = jnp.concatenate([dst, pad_sct]).reshape(NCHUNKS_PAD, CHUNK)
    dego, degi = _deg_call(srcp_d, dstp_d)
    xs = _xs_call(x, dego)
    aggp = _agg_call(srcp_a, dstp_a, xs)
    z = _z_call(aggp, W, degi, b.reshape(1, H))
    recon = _recon_call(z)
    return (recon, z)


# speedup vs baseline: 1.6835x; 1.6835x over previous
import functools

import jax
import jax.numpy as jnp
from jax import lax
from jax.experimental import pallas as pl
from jax.experimental.pallas import tpu as pltpu
from jax.experimental.pallas import tpu_sc as plsc

N = 10000
E = 160000
D = 128
H = 32

NC = 2
NS = 16
NW = NC * NS
CHUNK = 128
CPT = -(-E // (CHUNK * NW))
NCHUNKS_PAD = CPT * NW
E_PAD = NCHUNKS_PAD * CHUNK
N8 = N + 8
NZ = N + 128
NPAD = 10016
STRIPE = 624
TAIL = N - NS * STRIPE

_mesh = plsc.VectorSubcoreMesh(core_axis_name="c", subcore_axis_name="s")


@functools.partial(
    pl.kernel,
    out_type=(
        jax.ShapeDtypeStruct((NC, 1, NPAD), jnp.float32),
        jax.ShapeDtypeStruct((NC, 1, NPAD), jnp.float32),
    ),
    mesh=_mesh,
    scratch_types=[
        pltpu.VMEM((CPT, CHUNK), jnp.int32),
        pltpu.VMEM((CPT, CHUNK), jnp.int32),
        pltpu.VMEM((CHUNK,), jnp.float32),
        pltpu.VMEM((NPAD,), jnp.float32),
        pltpu.VMEM_SHARED((NPAD,), jnp.float32),
        pltpu.VMEM_SHARED((NPAD,), jnp.float32),
        pltpu.SemaphoreType.DMA,
        pltpu.SemaphoreType.DMA,
    ],
)
def _deg_call(src_hbm, dst_hbm, dego_hbm, degi_hbm, src_idx, dst_idx, ones_v,
              zeros_v, sh_dego, sh_degi, sem_a, sem_b):
    cid = lax.axis_index("c")
    sid = lax.axis_index("s")
    wid = cid * NS + sid

    pltpu.sync_copy(src_hbm.at[pl.ds(wid * CPT, CPT)], src_idx)
    pltpu.sync_copy(dst_hbm.at[pl.ds(wid * CPT, CPT)], dst_idx)

    for i in range(CHUNK // 16):
        ones_v[pl.ds(i * 16, 16)] = jnp.ones((16,), jnp.float32)

    @pl.when(sid == 0)
    def _init():
        def zbody(i, _):
            zeros_v[pl.ds(i * 16, 16)] = jnp.zeros((16,), jnp.float32)
            return ()
        lax.fori_loop(0, NPAD // 16, zbody, ())
        pltpu.sync_copy(zeros_v, sh_dego)
        pltpu.sync_copy(zeros_v, sh_degi)

    plsc.subcore_barrier()

    def dbody(j, _):
        pltpu.sync_copy(ones_v, sh_dego.at[src_idx.at[j]], add=True)
        pltpu.sync_copy(ones_v, sh_degi.at[dst_idx.at[j]], add=True)
        return ()

    lax.fori_loop(0, CPT, dbody, ())

    plsc.subcore_barrier()

    @pl.when(sid == 0)
    def _writeout():
        pltpu.sync_copy(sh_dego, dego_hbm.at[cid, 0])
        pltpu.sync_copy(sh_degi, degi_hbm.at[cid, 0])


@functools.partial(
    pl.kernel,
    out_type=jax.ShapeDtypeStruct((NC, N, D), jnp.float32),
    mesh=_mesh,
    scratch_types=[
        pltpu.VMEM((CPT, CHUNK), jnp.int32),
        pltpu.VMEM((CPT, CHUNK), jnp.int32),
        pltpu.VMEM((CHUNK, D), jnp.float32),
        pltpu.VMEM((CHUNK, D), jnp.float32),
        pltpu.VMEM((16, D), jnp.float32),
        pltpu.VMEM_SHARED((N8, D), jnp.float32),
        pltpu.SemaphoreType.DMA,
        pltpu.SemaphoreType.DMA,
        pltpu.SemaphoreType.DMA,
        pltpu.SemaphoreType.DMA,
    ],
)
def _agg_call(src_hbm, dst_hbm, xs_hbm, agg_hbm, src_idx, dst_idx, rows0,
              rows1, zrow_v, sh_agg, gsem0, gsem1, ssem0, ssem1):
    cid = lax.axis_index("c")
    sid = lax.axis_index("s")
    wid = cid * NS + sid

    pltpu.sync_copy(src_hbm.at[pl.ds(wid * CPT, CPT)], src_idx)
    pltpu.sync_copy(dst_hbm.at[pl.ds(wid * CPT, CPT)], dst_idx)

    def zbody(i, _):
        for j in range(D // 16):
            zrow_v[i, pl.ds(j * 16, 16)] = jnp.zeros((16,), jnp.float32)
        return ()
    lax.fori_loop(0, 16, zbody, ())

    def zcopy(k, _):
        pltpu.sync_copy(zrow_v, sh_agg.at[pl.ds(sid * STRIPE + k * 16, 16)])
        return ()
    lax.fori_loop(0, STRIPE // 16, zcopy, ())

    @pl.when(sid == 0)
    def _init_tail():
        pltpu.sync_copy(zrow_v, sh_agg.at[pl.ds(NS * STRIPE, TAIL)])

    plsc.subcore_barrier()

    rows = (rows0, rows1)
    gsem = (gsem0, gsem1)
    gd = [None, None]

    gd[0] = pltpu.async_copy(xs_hbm.at[src_idx.at[0]], rows[0], gsem[0])
    for j in range(CPT):
        b = j % 2
        nb = (j + 1) % 2
        if j + 1 < CPT:
            gd[nb] = pltpu.async_copy(
                xs_hbm.at[src_idx.at[j + 1]], rows[nb], gsem[nb])
        gd[b].wait()
        pltpu.sync_copy(rows[b], sh_agg.at[dst_idx.at[j]], add=True)

    plsc.subcore_barrier()

    row0 = sid * STRIPE
    pltpu.sync_copy(sh_agg.at[pl.ds(row0, STRIPE)],
                    agg_hbm.at[cid, pl.ds(row0, STRIPE)])

    @pl.when(sid == 0)
    def _writeout_tail():
        pltpu.sync_copy(sh_agg.at[pl.ds(NS * STRIPE, TAIL)],
                        agg_hbm.at[cid, pl.ds(NS * STRIPE, TAIL)])


def _xs_body(x_ref, dego_ref, xs_ref):
    d = dego_ref[0, 0, pl.ds(0, N)] + dego_ref[1, 0, pl.ds(0, N)]
    norm = jnp.where(d > 0, lax.rsqrt(jnp.maximum(d, 1.0)), 0.0)
    xs_ref[pl.ds(0, N), :] = x_ref[...] * norm[:, None]
    xs_ref[pl.ds(N, 128), :] = jnp.zeros((128, D), jnp.float32)


def _xs_call(x, dego):
    return pl.pallas_call(
        _xs_body,
        out_shape=jax.ShapeDtypeStruct((NZ, D), jnp.float32),
    )(x, dego)


def _z_body(agg_ref, w_ref, degi_ref, b_ref, z_ref):
    a = agg_ref[0] + agg_ref[1]
    d = degi_ref[0, 0, pl.ds(0, N)] + degi_ref[1, 0, pl.ds(0, N)]
    norm = jnp.where(d > 0, lax.rsqrt(jnp.maximum(d, 1.0)), 0.0)
    zw = jnp.dot(a, w_ref[...], preferred_element_type=jnp.float32)
    z_ref[...] = zw * norm[:, None] + b_ref[0, :][None, :]


def _z_call(aggp, W, degi, b2d):
    return pl.pallas_call(
        _z_body,
        out_shape=jax.ShapeDtypeStruct((N, H), jnp.float32),
    )(aggp, W, degi, b2d)


BI = 256
NBI = (N + BI - 1) // BI


def _recon_body(zi_ref, zall_ref, out_ref):
    out_ref[...] = lax.dot_general(
        zi_ref[...], zall_ref[...],
        dimension_numbers=(((1,), (1,)), ((), ())),
        preferred_element_type=jnp.float32,
    )


def _recon_call(z):
    return pl.pallas_call(
        _recon_body,
        grid=(NBI,),
        in_specs=[
            pl.BlockSpec((BI, H), lambda i: (i, 0)),
            pl.BlockSpec((N, H), lambda i: (0, 0)),
        ],
        out_specs=pl.BlockSpec((BI, N), lambda i: (i, 0)),
        out_shape=jax.ShapeDtypeStruct((N, N), jnp.float32),
    )(z, z)


def kernel(x, edge_index, W, b):
    npad = E_PAD - E
    pad_deg = N + (jnp.arange(npad, dtype=jnp.int32) % 16)
    pad_gat = N + (jnp.arange(npad, dtype=jnp.int32) % 128)
    pad_sct = jnp.arange(npad, dtype=jnp.int32) % N
    src = edge_index[0]
    dst = edge_index[1]
    srcp_d = jnp.concatenate([src, pad_deg]).reshape(NCHUNKS_PAD, CHUNK)
    dstp_d = jnp.concatenate([dst, pad_deg]).reshape(NCHUNKS_PAD, CHUNK)
    srcp_a = jnp.concatenate([src, pad_gat]).reshape(NCHUNKS_PAD, CHUNK)
    dstp_a = jnp.concatenate([dst, pad_sct]).reshape(NCHUNKS_PAD, CHUNK)
    dego, degi = _deg_call(srcp_d, dstp_d)
    xs = _xs_call(x, dego)
    aggp = _agg_call(srcp_a, dstp_a, xs)
    z = _z_call(aggp, W, degi, b.reshape(1, H))
    recon = _recon_call(z)
    return (recon, z)
